# Initial kernel scaffold; baseline (speedup 1.0000x reference)
import jax
import jax.numpy as jnp
from jax.experimental import pallas as pl


def kernel(x, edge_index, Wl1, bl1, Wr1, Wl2, bl2, Wr2, W3, b3, W4, b4):
    raise NotImplementedError("write your pallas kernel here")


# speedup vs baseline: 3.2212x; 3.2212x over previous
import functools

import jax
import jax.numpy as jnp
from jax import lax
from jax.experimental import pallas as pl
from jax.experimental.pallas import tpu as pltpu
from jax.experimental.pallas import tpu_sc as plsc

N = 10000
D = 128
E = 320000
NC = 2
NS = 16
NW = NC * NS
C = 128
CHUNKS = 80
EW = C * CHUNKS
E_PAD = EW * NW
N_PAD = 10240
ZROWS = N_PAD // NS
DEGW = 16
IB = 8

def _deg_body(dst_hbm, deg_hbm,
              hist_v, didx_v, acc_v, tmp_v, col_v, sh, sem):
    c = lax.axis_index("c")
    s = lax.axis_index("s")
    w = c * NS + s
    ones16 = jnp.ones((16,), jnp.float32)
    zeros16 = jnp.zeros((16,), jnp.float32)

    def z(i, carry):
        hist_v[pl.ds(i * 16, 16)] = zeros16
        return carry

    lax.fori_loop(0, N_PAD // 16, z, 0)

    def block(b, carry):
        pltpu.sync_copy(dst_hbm.at[pl.ds(w * CHUNKS + b * IB, IB)], didx_v)

        def chunk(k, carry2):
            def sub(j, carry3):
                idx16 = didx_v[k, pl.ds(j * 16, 16)]
                plsc.addupdate_scatter(hist_v, [idx16], ones16)
                return carry3

            return lax.fori_loop(0, C // 16, sub, carry2)

        return lax.fori_loop(0, IB, chunk, carry)

    lax.fori_loop(0, CHUNKS // IB, block, 0)
    pltpu.sync_copy(hist_v, sh.at[s])
    plsc.subcore_barrier()

    def z2(i, carry):
        acc_v[pl.ds(i * 16, 16)] = zeros16
        return carry

    lax.fori_loop(0, ZROWS // 16, z2, 0)

    def red(t, carry):
        pltpu.sync_copy(sh.at[t, pl.ds(s * ZROWS, ZROWS)], tmp_v)

        def add(i, carry2):
            acc_v[pl.ds(i * 16, 16)] = (acc_v[pl.ds(i * 16, 16)]
                                        + tmp_v[pl.ds(i * 16, 16)])
            return carry2

        return lax.fori_loop(0, ZROWS // 16, add, carry)

    lax.fori_loop(0, NS, red, 0)

    def zc(i, carry):
        col_v[i] = zeros16
        return carry

    lax.fori_loop(0, ZROWS, zc, 0)
    lanes = lax.iota(jnp.int32, 16)

    def tp(i, carry):
        vals = acc_v[pl.ds(i * 16, 16)]
        plsc.store_scatter(col_v, [i * 16 + lanes, lanes * 0], vals)
        return carry

    lax.fori_loop(0, ZROWS // 16, tp, 0)
    for ci in range(NC):
        @pl.when(c == ci)
        def _copy_out(ci=ci):
            pltpu.sync_copy(col_v, deg_hbm.at[ci, pl.ds(s * ZROWS, ZROWS)])


@functools.cache
def _get_deg():
    return pl.kernel(
        _deg_body,
        out_type=[jax.ShapeDtypeStruct((NC, N_PAD, DEGW), jnp.float32)],
        mesh=plsc.VectorSubcoreMesh(core_axis_name="c", subcore_axis_name="s"),
        compiler_params=pltpu.CompilerParams(needs_layout_passes=False),
        scratch_types=[
            pltpu.VMEM((N_PAD,), jnp.float32),
            pltpu.VMEM((IB, C), jnp.int32),
            pltpu.VMEM((ZROWS,), jnp.float32),
            pltpu.VMEM((ZROWS,), jnp.float32),
            pltpu.VMEM((ZROWS, DEGW), jnp.float32),
            pltpu.VMEM_SHARED((NS, N_PAD), jnp.float32),
            pltpu.SemaphoreType.DMA,
        ],
    )


def _segsum_body(y_hbm, src_hbm, dst_hbm, z_hbm,
                 out_hbm,
                 acc_sh, sidx_v, didx_v, rows_v, sem):
    c = lax.axis_index("c")
    s = lax.axis_index("s")
    w = c * NS + s
    pltpu.sync_copy(z_hbm, rows_v)

    def zloop(j, carry):
        pltpu.sync_copy(rows_v, acc_sh.at[pl.ds(s * ZROWS + j * C, C)])
        return carry

    lax.fori_loop(0, ZROWS // C, zloop, 0)
    plsc.subcore_barrier()

    def block(b, carry):
        pltpu.sync_copy(src_hbm.at[pl.ds(w * CHUNKS + b * IB, IB)], sidx_v)
        pltpu.sync_copy(dst_hbm.at[pl.ds(w * CHUNKS + b * IB, IB)], didx_v)

        def chunk(k, carry2):
            pltpu.async_copy(y_hbm.at[sidx_v.at[k]], rows_v, sem).wait()
            pltpu.sync_copy(rows_v, acc_sh.at[didx_v.at[k]], add=True)
            return carry2

        return lax.fori_loop(0, IB, chunk, carry)

    lax.fori_loop(0, CHUNKS // IB, block, 0)
    plsc.subcore_barrier()

    for ci in range(NC):
        @pl.when(c == ci)
        def _copy_out(ci=ci):
            def oloop(j, carry):
                pltpu.sync_copy(acc_sh.at[pl.ds(s * ZROWS + j * C, C)], rows_v)
                pltpu.sync_copy(rows_v,
                                out_hbm.at[ci, pl.ds(s * ZROWS + j * C, C)])
                return carry

            lax.fori_loop(0, ZROWS // C, oloop, 0)


@functools.cache
def _get_segsum():
    return pl.kernel(
        _segsum_body,
        out_type=[jax.ShapeDtypeStruct((NC, N_PAD, D), jnp.float32)],
        mesh=plsc.VectorSubcoreMesh(core_axis_name="c", subcore_axis_name="s"),
        scratch_types=[
            pltpu.VMEM_SHARED((N_PAD, D), jnp.float32),
            pltpu.VMEM((IB, C), jnp.int32),
            pltpu.VMEM((IB, C), jnp.int32),
            pltpu.VMEM((C, D), jnp.float32),
            pltpu.SemaphoreType.DMA,
        ],
    )


BM = 1000


def _tc_pre_body(x_ref, w_ref, b_ref, yl_ref, yr_ref):
    y = jnp.dot(x_ref[...], w_ref[...], preferred_element_type=jnp.float32)
    yl_ref[...] = y[:, :D]
    yr_ref[...] = y[:, D:] + b_ref[...]


def _tc_pre(x, wcat, b):
    return pl.pallas_call(
        _tc_pre_body,
        grid=(N // BM,),
        in_specs=[pl.BlockSpec((BM, D), lambda i: (i, 0)),
                  pl.BlockSpec((D, 2 * D), lambda i: (0, 0)),
                  pl.BlockSpec((1, D), lambda i: (0, 0))],
        out_specs=[pl.BlockSpec((BM, D), lambda i: (i, 0)),
                   pl.BlockSpec((BM, D), lambda i: (i, 0))],
        out_shape=[jax.ShapeDtypeStruct((N, D), jnp.float32),
                   jax.ShapeDtypeStruct((N, D), jnp.float32)],
    )(x, wcat, b)


def _tc_mid_body(s_ref, deg_ref, yr_ref, w_ref, b_ref, yl2_ref, yr2_ref):
    ssum = s_ref[0] + s_ref[1]
    deg = deg_ref[0, :, 0:1] + deg_ref[1, :, 0:1]
    inv = 1.0 / jnp.maximum(deg, 1.0)
    h = jnp.maximum(ssum * inv + yr_ref[...], 0.0)
    y = jnp.dot(h, w_ref[...], preferred_element_type=jnp.float32)
    yl2_ref[...] = y[:, :D]
    yr2_ref[...] = y[:, D:] + b_ref[...]


def _tc_mid(s1, degp, y1r, wcat, b):
    return pl.pallas_call(
        _tc_mid_body,
        grid=(N // BM,),
        in_specs=[pl.BlockSpec((NC, BM, D), lambda i: (0, i, 0)),
                  pl.BlockSpec((NC, BM, DEGW), lambda i: (0, i, 0)),
                  pl.BlockSpec((BM, D), lambda i: (i, 0)),
                  pl.BlockSpec((D, 2 * D), lambda i: (0, 0)),
                  pl.BlockSpec((1, D), lambda i: (0, 0))],
        out_specs=[pl.BlockSpec((BM, D), lambda i: (i, 0)),
                   pl.BlockSpec((BM, D), lambda i: (i, 0))],
        out_shape=[jax.ShapeDtypeStruct((N, D), jnp.float32),
                   jax.ShapeDtypeStruct((N, D), jnp.float32)],
    )(s1, degp, y1r, wcat, b)


def _tc_post_body(s_ref, deg_ref, yr_ref, w3_ref, b3_ref, w4_ref, b4_ref,
                  out_ref):
    ssum = s_ref[0] + s_ref[1]
    deg = deg_ref[0, :, 0:1] + deg_ref[1, :, 0:1]
    inv = 1.0 / jnp.maximum(deg, 1.0)
    h = jnp.maximum(ssum * inv + yr_ref[...], 0.0)
    h = jnp.dot(h, w3_ref[...], preferred_element_type=jnp.float32) + b3_ref[...]
    h = jnp.dot(h, w4_ref[...], preferred_element_type=jnp.float32) + b4_ref[...]
    m = jnp.max(h, axis=1, keepdims=True)
    lse = jnp.log(jnp.sum(jnp.exp(h - m), axis=1, keepdims=True)) + m
    out_ref[...] = h - lse


def _tc_post(s2, degp, y2r, w3, b3, w4, b4):
    return pl.pallas_call(
        _tc_post_body,
        grid=(N // BM,),
        in_specs=[pl.BlockSpec((NC, BM, D), lambda i: (0, i, 0)),
                  pl.BlockSpec((NC, BM, DEGW), lambda i: (0, i, 0)),
                  pl.BlockSpec((BM, D), lambda i: (i, 0)),
                  pl.BlockSpec((D, D), lambda i: (0, 0)),
                  pl.BlockSpec((1, D), lambda i: (0, 0)),
                  pl.BlockSpec((D, D), lambda i: (0, 0)),
                  pl.BlockSpec((1, D), lambda i: (0, 0))],
        out_specs=pl.BlockSpec((BM, D), lambda i: (i, 0)),
        out_shape=jax.ShapeDtypeStruct((N, D), jnp.float32),
    )(s2, degp, y2r, w3, b3, w4, b4)


def kernel(x, edge_index, Wl1, bl1, Wr1, Wl2, bl2, Wr2, W3, b3, W4, b4):
    src = edge_index[0]
    dst = edge_index[1]
    pad = E_PAD - E
    src2d = jnp.concatenate([src, jnp.zeros((pad,), jnp.int32)]).reshape(
        NW * CHUNKS, C)
    dst2d = jnp.concatenate([dst, jnp.full((pad,), N, jnp.int32)]).reshape(
        NW * CHUNKS, C)
    zeros_b = jnp.zeros((C, D), jnp.float32)
    wcat1 = jnp.concatenate([Wl1.T, Wr1.T], axis=1)
    wcat2 = jnp.concatenate([Wl2.T, Wr2.T], axis=1)

    degp, = _get_deg()(dst2d)
    y1l, y1r = _tc_pre(x, wcat1, bl1.reshape(1, D))
    s1, = _get_segsum()(y1l, src2d, dst2d, zeros_b)
    y2l, y2r = _tc_mid(s1, degp, y1r, wcat2, bl2.reshape(1, D))
    s2, = _get_segsum()(y2l, src2d, dst2d, zeros_b)
    return _tc_post(s2, degp, y2r, W3.T, b3.reshape(1, D), W4.T,
                    b4.reshape(1, D))


# speedup vs baseline: 3.5027x; 1.0874x over previous
import functools

import jax
import jax.numpy as jnp
from jax import lax
from jax.experimental import pallas as pl
from jax.experimental.pallas import tpu as pltpu
from jax.experimental.pallas import tpu_sc as plsc

N = 10000
D = 128
E = 320000
NC = 2
NS = 16
NW = NC * NS
C = 128
CHUNKS = 80
EW = C * CHUNKS
E_PAD = EW * NW
N_PAD = 10240
ZROWS = N_PAD // NS
DEGW = 16
IB = 8

def _deg_body(dst_hbm, deg_hbm,
              hist_v, didx_v, acc_v, tmp_v, col_v, sh, sem):
    c = lax.axis_index("c")
    s = lax.axis_index("s")
    w = c * NS + s
    ones16 = jnp.ones((16,), jnp.float32)
    zeros16 = jnp.zeros((16,), jnp.float32)

    def z(i, carry):
        hist_v[pl.ds(i * 16, 16)] = zeros16
        return carry

    lax.fori_loop(0, N_PAD // 16, z, 0)

    def block(b, carry):
        pltpu.sync_copy(dst_hbm.at[pl.ds(w * CHUNKS + b * IB, IB)], didx_v)

        def chunk(k, carry2):
            def sub(j, carry3):
                idx16 = didx_v[k, pl.ds(j * 16, 16)]
                plsc.addupdate_scatter(hist_v, [idx16], ones16)
                return carry3

            return lax.fori_loop(0, C // 16, sub, carry2)

        return lax.fori_loop(0, IB, chunk, carry)

    lax.fori_loop(0, CHUNKS // IB, block, 0)
    pltpu.sync_copy(hist_v, sh.at[s])
    plsc.subcore_barrier()

    def z2(i, carry):
        acc_v[pl.ds(i * 16, 16)] = zeros16
        return carry

    lax.fori_loop(0, ZROWS // 16, z2, 0)

    def red(t, carry):
        pltpu.sync_copy(sh.at[t, pl.ds(s * ZROWS, ZROWS)], tmp_v)

        def add(i, carry2):
            acc_v[pl.ds(i * 16, 16)] = (acc_v[pl.ds(i * 16, 16)]
                                        + tmp_v[pl.ds(i * 16, 16)])
            return carry2

        return lax.fori_loop(0, ZROWS // 16, add, carry)

    lax.fori_loop(0, NS, red, 0)

    def zc(i, carry):
        col_v[i] = zeros16
        return carry

    lax.fori_loop(0, ZROWS, zc, 0)
    lanes = lax.iota(jnp.int32, 16)

    def tp(i, carry):
        vals = acc_v[pl.ds(i * 16, 16)]
        plsc.store_scatter(col_v, [i * 16 + lanes, lanes * 0], vals)
        return carry

    lax.fori_loop(0, ZROWS // 16, tp, 0)
    for ci in range(NC):
        @pl.when(c == ci)
        def _copy_out(ci=ci):
            pltpu.sync_copy(col_v, deg_hbm.at[ci, pl.ds(s * ZROWS, ZROWS)])


@functools.cache
def _get_deg():
    return pl.kernel(
        _deg_body,
        out_type=[jax.ShapeDtypeStruct((NC, N_PAD, DEGW), jnp.float32)],
        mesh=plsc.VectorSubcoreMesh(core_axis_name="c", subcore_axis_name="s"),
        compiler_params=pltpu.CompilerParams(needs_layout_passes=False),
        scratch_types=[
            pltpu.VMEM((N_PAD,), jnp.float32),
            pltpu.VMEM((IB, C), jnp.int32),
            pltpu.VMEM((ZROWS,), jnp.float32),
            pltpu.VMEM((ZROWS,), jnp.float32),
            pltpu.VMEM((ZROWS, DEGW), jnp.float32),
            pltpu.VMEM_SHARED((NS, N_PAD), jnp.float32),
            pltpu.SemaphoreType.DMA,
        ],
    )


def _segsum_body(y_hbm, src_hbm, dst_hbm,
                 out_hbm,
                 acc_sh, sidx_v, didx_v, rows_v, sem):
    c = lax.axis_index("c")
    s = lax.axis_index("s")
    w = c * NS + s
    zeros16 = jnp.zeros((16,), jnp.float32)

    def zrow(i, carry):
        for l in range(D // 16):
            rows_v[0, i, pl.ds(l * 16, 16)] = zeros16
        return carry

    lax.fori_loop(0, C, zrow, 0)

    def zloop(j, carry):
        pltpu.sync_copy(rows_v.at[0], acc_sh.at[pl.ds(s * ZROWS + j * C, C)])
        return carry

    lax.fori_loop(0, ZROWS // C, zloop, 0)
    plsc.subcore_barrier()

    def block(b, carry):
        pltpu.sync_copy(src_hbm.at[pl.ds(w * CHUNKS + b * IB, IB)], sidx_v)
        pltpu.sync_copy(dst_hbm.at[pl.ds(w * CHUNKS + b * IB, IB)], didx_v)
        cp = pltpu.async_copy(y_hbm.at[sidx_v.at[0]], rows_v.at[0], sem)
        for k in range(IB):
            cp.wait()
            if k + 1 < IB:
                cp = pltpu.async_copy(y_hbm.at[sidx_v.at[k + 1]],
                                      rows_v.at[(k + 1) % 2], sem)
            pltpu.sync_copy(rows_v.at[k % 2], acc_sh.at[didx_v.at[k]],
                            add=True)
        return carry

    lax.fori_loop(0, CHUNKS // IB, block, 0)
    plsc.subcore_barrier()

    for ci in range(NC):
        @pl.when(c == ci)
        def _copy_out(ci=ci):
            def oloop(j, carry):
                pltpu.sync_copy(acc_sh.at[pl.ds(s * ZROWS + j * C, C)],
                                rows_v.at[0])
                pltpu.sync_copy(rows_v.at[0],
                                out_hbm.at[ci, pl.ds(s * ZROWS + j * C, C)])
                return carry

            lax.fori_loop(0, ZROWS // C, oloop, 0)


@functools.cache
def _get_segsum():
    return pl.kernel(
        _segsum_body,
        out_type=[jax.ShapeDtypeStruct((NC, N_PAD, D), jnp.float32)],
        mesh=plsc.VectorSubcoreMesh(core_axis_name="c", subcore_axis_name="s"),
        scratch_types=[
            pltpu.VMEM_SHARED((N_PAD, D), jnp.float32),
            pltpu.VMEM((IB, C), jnp.int32),
            pltpu.VMEM((IB, C), jnp.int32),
            pltpu.VMEM((2, C, D), jnp.float32),
            pltpu.SemaphoreType.DMA,
        ],
    )


BM = 1000


def _tc_pre_body(x_ref, w_ref, b_ref, yl_ref, yr_ref):
    y = jnp.dot(x_ref[...], w_ref[...], preferred_element_type=jnp.float32)
    yl_ref[...] = y[:, :D]
    yr_ref[...] = y[:, D:] + b_ref[...]


def _tc_pre(x, wcat, b):
    return pl.pallas_call(
        _tc_pre_body,
        grid=(N // BM,),
        in_specs=[pl.BlockSpec((BM, D), lambda i: (i, 0)),
                  pl.BlockSpec((D, 2 * D), lambda i: (0, 0)),
                  pl.BlockSpec((1, D), lambda i: (0, 0))],
        out_specs=[pl.BlockSpec((BM, D), lambda i: (i, 0)),
                   pl.BlockSpec((BM, D), lambda i: (i, 0))],
        out_shape=[jax.ShapeDtypeStruct((N, D), jnp.float32),
                   jax.ShapeDtypeStruct((N, D), jnp.float32)],
    )(x, wcat, b)


def _tc_mid_body(s_ref, deg_ref, yr_ref, w_ref, b_ref, yl2_ref, yr2_ref):
    ssum = s_ref[0] + s_ref[1]
    deg = deg_ref[0, :, 0:1] + deg_ref[1, :, 0:1]
    inv = 1.0 / jnp.maximum(deg, 1.0)
    h = jnp.maximum(ssum * inv + yr_ref[...], 0.0)
    y = jnp.dot(h, w_ref[...], preferred_element_type=jnp.float32)
    yl2_ref[...] = y[:, :D]
    yr2_ref[...] = y[:, D:] + b_ref[...]


def _tc_mid(s1, degp, y1r, wcat, b):
    return pl.pallas_call(
        _tc_mid_body,
        grid=(N // BM,),
        in_specs=[pl.BlockSpec((NC, BM, D), lambda i: (0, i, 0)),
                  pl.BlockSpec((NC, BM, DEGW), lambda i: (0, i, 0)),
                  pl.BlockSpec((BM, D), lambda i: (i, 0)),
                  pl.BlockSpec((D, 2 * D), lambda i: (0, 0)),
                  pl.BlockSpec((1, D), lambda i: (0, 0))],
        out_specs=[pl.BlockSpec((BM, D), lambda i: (i, 0)),
                   pl.BlockSpec((BM, D), lambda i: (i, 0))],
        out_shape=[jax.ShapeDtypeStruct((N, D), jnp.float32),
                   jax.ShapeDtypeStruct((N, D), jnp.float32)],
    )(s1, degp, y1r, wcat, b)


def _tc_post_body(s_ref, deg_ref, yr_ref, w3_ref, b3_ref, w4_ref, b4_ref,
                  out_ref):
    ssum = s_ref[0] + s_ref[1]
    deg = deg_ref[0, :, 0:1] + deg_ref[1, :, 0:1]
    inv = 1.0 / jnp.maximum(deg, 1.0)
    h = jnp.maximum(ssum * inv + yr_ref[...], 0.0)
    h = jnp.dot(h, w3_ref[...], preferred_element_type=jnp.float32) + b3_ref[...]
    h = jnp.dot(h, w4_ref[...], preferred_element_type=jnp.float32) + b4_ref[...]
    m = jnp.max(h, axis=1, keepdims=True)
    lse = jnp.log(jnp.sum(jnp.exp(h - m), axis=1, keepdims=True)) + m
    out_ref[...] = h - lse


def _tc_post(s2, degp, y2r, w3, b3, w4, b4):
    return pl.pallas_call(
        _tc_post_body,
        grid=(N // BM,),
        in_specs=[pl.BlockSpec((NC, BM, D), lambda i: (0, i, 0)),
                  pl.BlockSpec((NC, BM, DEGW), lambda i: (0, i, 0)),
                  pl.BlockSpec((BM, D), lambda i: (i, 0)),
                  pl.BlockSpec((D, D), lambda i: (0, 0)),
                  pl.BlockSpec((1, D), lambda i: (0, 0)),
                  pl.BlockSpec((D, D), lambda i: (0, 0)),
                  pl.BlockSpec((1, D), lambda i: (0, 0))],
        out_specs=pl.BlockSpec((BM, D), lambda i: (i, 0)),
        out_shape=jax.ShapeDtypeStruct((N, D), jnp.float32),
    )(s2, degp, y2r, w3, b3, w4, b4)


def kernel(x, edge_index, Wl1, bl1, Wr1, Wl2, bl2, Wr2, W3, b3, W4, b4):
    src = edge_index[0]
    dst = edge_index[1]
    pad = E_PAD - E
    src2d = jnp.concatenate([src, jnp.zeros((pad,), jnp.int32)]).reshape(
        NW * CHUNKS, C)
    dst2d = jnp.concatenate([dst, jnp.full((pad,), N, jnp.int32)]).reshape(
        NW * CHUNKS, C)
    wcat1 = jnp.concatenate([Wl1.T, Wr1.T], axis=1)
    wcat2 = jnp.concatenate([Wl2.T, Wr2.T], axis=1)

    degp, = _get_deg()(dst2d)
    y1l, y1r = _tc_pre(x, wcat1, bl1.reshape(1, D))
    s1, = _get_segsum()(y1l, src2d, dst2d)
    y2l, y2r = _tc_mid(s1, degp, y1r, wcat2, bl2.reshape(1, D))
    s2, = _get_segsum()(y2l, src2d, dst2d)
    return _tc_post(s2, degp, y2r, W3.T, b3.reshape(1, D), W4.T,
                    b4.reshape(1, D))
